# baseline (speedup 1.0000x reference)
import functools

import jax
import jax.numpy as jnp
from jax import lax
from jax.experimental import pallas as pl
from jax.experimental.pallas import tpu as pltpu
from jax.experimental.pallas import tpu_sc as plsc

_E = 16
_TOPK = 2
_BM = 256
_NC = 2
_NS = 16
_NW = _NC * _NS
_L = 16


def _router_body(x_ref, wr_ref, br_ref, pos_ref, wk_ref, bmap_ref, xidx_ref,
                 nact_ref, NB):
    T, E = x_ref.shape[0], wr_ref.shape[1]
    logits = jnp.dot(x_ref[...], wr_ref[...],
                     preferred_element_type=jnp.float32) + br_ref[...]
    m = jnp.max(logits, axis=1, keepdims=True)
    p = jnp.exp(logits - m)
    Z = jnp.sum(p, axis=1, keepdims=True)
    ii = lax.broadcasted_iota(jnp.int32, (T, E), 1)
    e0 = jnp.min(jnp.where(logits == m, ii, E), axis=1, keepdims=True)
    oh0 = ii == e0
    logits1 = jnp.where(oh0, jnp.float32(-1e30), logits)
    l1 = jnp.max(logits1, axis=1, keepdims=True)
    e1 = jnp.min(jnp.where(logits1 == l1, ii, E), axis=1, keepdims=True)
    oh1 = ii == e1
    w0 = 1.0 / Z
    w1 = jnp.exp(l1 - m) / Z

    oh = jnp.concatenate([oh0, oh1], axis=0).astype(jnp.int32)
    inc = oh
    k = 1
    while k < 2 * T:
        inc = inc + jnp.concatenate(
            [jnp.zeros((k, E), jnp.int32), inc[:-k]], axis=0)
        k *= 2
    excl = inc - oh
    rank = jnp.sum(excl * oh, axis=1, keepdims=True)
    counts = jnp.sum(oh, axis=0, keepdims=True)
    blocks = (counts + (_BM - 1)) // _BM
    binc = blocks
    k = 1
    while k < E:
        binc = binc + jnp.concatenate(
            [jnp.zeros((1, k), jnp.int32), binc[:, :-k]], axis=1)
        k *= 2
    bstart = binc - blocks
    seg = _BM * bstart
    pos = jnp.sum(seg * oh, axis=1, keepdims=True) + rank
    nact = jnp.sum(blocks)

    g = lax.broadcasted_iota(jnp.int32, (1, NB), 1)
    acc = jnp.zeros((1, NB), jnp.int32)
    for e in range(E):
        acc = acc + (g >= bstart[0, e]).astype(jnp.int32)
    bmap_raw = acc - 1
    last = jnp.sum(jnp.where(g == nact - 1, bmap_raw, 0), axis=1,
                   keepdims=True)
    pos_ref[...] = pos
    wk_ref[...] = jnp.concatenate([w0, w1], axis=0)
    bmap_ref[...] = jnp.where(g < nact, bmap_raw, last)
    xidx_ref[...] = jnp.where(g < nact, g, nact - 1)
    nact_ref[...] = jnp.full((1, 1), 0, jnp.int32) + nact


def _router(xf, Wr, br, NB):
    T = xf.shape[0]
    return pl.pallas_call(
        functools.partial(_router_body, NB=NB),
        out_shape=[
            jax.ShapeDtypeStruct((_TOPK * T, 1), jnp.int32),
            jax.ShapeDtypeStruct((_TOPK * T, 1), jnp.float32),
            jax.ShapeDtypeStruct((1, NB), jnp.int32),
            jax.ShapeDtypeStruct((1, NB), jnp.int32),
            jax.ShapeDtypeStruct((1, 1), jnp.int32),
        ],
    )(xf, Wr, br[None, :])


def _ffn_body(nact_ref, xidx_ref, bmap_ref, x_ref, w1_ref, b1_ref, w2_ref,
              b2_ref, wcol_ref, o_ref):
    g = pl.program_id(0)

    @pl.when(g < nact_ref[0])
    def _():
        hmid = jnp.dot(x_ref[...].astype(jnp.bfloat16), w1_ref[0],
                       preferred_element_type=jnp.float32)
        hmid = jnp.maximum(hmid + b1_ref[0], 0.0)
        y = jnp.dot(hmid.astype(jnp.bfloat16), w2_ref[0],
                    preferred_element_type=jnp.float32)
        y = y + b2_ref[0]
        o_ref[...] = y * wcol_ref[...]


def _grouped_ffn(nact, xidx, bmap, xs, W1, b1, W2, b2, w_col, NB, P, H, F):
    grid_spec = pltpu.PrefetchScalarGridSpec(
        num_scalar_prefetch=3,
        grid=(NB,),
        in_specs=[
            pl.BlockSpec((_BM, H), lambda g, n, xi, bm: (xi[g], 0)),
            pl.BlockSpec((1, H, F), lambda g, n, xi, bm: (bm[g], 0, 0)),
            pl.BlockSpec((1, 1, F), lambda g, n, xi, bm: (bm[g], 0, 0)),
            pl.BlockSpec((1, F, H), lambda g, n, xi, bm: (bm[g], 0, 0)),
            pl.BlockSpec((1, 1, H), lambda g, n, xi, bm: (bm[g], 0, 0)),
            pl.BlockSpec((_BM, 1), lambda g, n, xi, bm: (xi[g], 0)),
        ],
        out_specs=pl.BlockSpec((_BM, H), lambda g, n, xi, bm: (xi[g], 0)),
    )
    return pl.pallas_call(
        _ffn_body,
        grid_spec=grid_spec,
        out_shape=jax.ShapeDtypeStruct((P, H), jnp.float32),
    )(nact, xidx, bmap, xs, W1, b1, W2, b2, w_col)


def _dispatch(xf, tok_sorted, P, H):
    rows_pw = P // _NW
    CH = 64
    NCH = rows_pw // CH
    mesh = plsc.VectorSubcoreMesh(core_axis_name="c", subcore_axis_name="s")

    @functools.partial(
        pl.kernel, mesh=mesh,
        out_type=jax.ShapeDtypeStruct((P, H), jnp.float32),
        scratch_types=(
            [pltpu.VMEM((CH,), jnp.int32) for _ in range(NCH)] +
            [pltpu.VMEM((CH, H), jnp.float32),
             pltpu.VMEM((CH, H), jnp.float32),
             pltpu.SemaphoreType.DMA,
             pltpu.SemaphoreType.DMA]
        ),
    )
    def k(xf_h, ts_h, xs_h, *rest):
        idx = rest[:NCH]
        buf0, buf1, sem0, sem1 = rest[NCH:]
        wid = lax.axis_index("s") * _NC + lax.axis_index("c")
        base = wid * rows_pw
        for c in range(NCH):
            pltpu.sync_copy(ts_h.at[pl.ds(base + c * CH, CH)], idx[c])
        bufs = (buf0, buf1)
        sems = (sem0, sem1)
        copies = [None, None]
        copies[0] = pltpu.async_copy(xf_h.at[idx[0]], buf0, sem0)
        for c in range(NCH):
            if c + 1 < NCH:
                copies[(c + 1) % 2] = pltpu.async_copy(
                    xf_h.at[idx[c + 1]], bufs[(c + 1) % 2],
                    sems[(c + 1) % 2])
            copies[c % 2].wait()
            pltpu.sync_copy(bufs[c % 2], xs_h.at[pl.ds(base + c * CH, CH)])

    return k(xf, tok_sorted)


def _combine(ys, p0, p1, T, H):
    tpw = T // _NW
    mesh = plsc.VectorSubcoreMesh(core_axis_name="c", subcore_axis_name="s")

    @functools.partial(
        pl.kernel, mesh=mesh,
        out_type=jax.ShapeDtypeStruct((T, H), jnp.float32),
        scratch_types=[
            pltpu.VMEM((tpw,), jnp.int32),
            pltpu.VMEM((tpw,), jnp.int32),
            pltpu.VMEM((tpw, H), jnp.float32),
            pltpu.VMEM((tpw, H), jnp.float32),
            pltpu.SemaphoreType.DMA,
            pltpu.SemaphoreType.DMA,
        ],
    )
    def k(ys_h, p0_h, p1_h, out_h, i0_v, i1_v, ba, bb, s0, s1):
        wid = lax.axis_index("s") * _NC + lax.axis_index("c")
        base = wid * tpw
        pltpu.sync_copy(p0_h.at[pl.ds(base, tpw)], i0_v)
        pltpu.sync_copy(p1_h.at[pl.ds(base, tpw)], i1_v)
        ca = pltpu.async_copy(ys_h.at[i0_v], ba, s0)
        cb = pltpu.async_copy(ys_h.at[i1_v], bb, s1)
        ca.wait()
        cb.wait()

        def addrow(r, carry):
            for j in range(H // _L):
                sl = pl.ds(j * _L, _L)
                ba[r, sl] = ba[r, sl] + bb[r, sl]
            return carry

        lax.fori_loop(0, tpw, addrow, 0)
        pltpu.sync_copy(ba, out_h.at[pl.ds(base, tpw)])

    return k(ys, p0, p1)


def kernel(x, Wr, br, W1, b1, W2, b2):
    b, s, h = x.shape
    T = b * s
    F = W1.shape[-1]
    E = Wr.shape[-1]
    xf = x.reshape(T, h)

    NB = (_TOPK * T) // _BM + E
    P = NB * _BM

    pos2d, wk2d, bmap2d, xidx2d, nact2d = _router(xf, Wr, br, NB)
    pos = pos2d.reshape(-1)
    w_flat = wk2d.reshape(-1)
    bmap = bmap2d.reshape(-1)
    xidx = xidx2d.reshape(-1)
    nact = nact2d.reshape(-1)

    tok = jnp.tile(jnp.arange(T, dtype=jnp.int32), _TOPK)
    tok_base = jnp.arange(P, dtype=jnp.int32) % T
    tok_sorted = tok_base.at[pos].set(tok)
    w_sorted = jnp.zeros((P,), x.dtype).at[pos].set(w_flat)

    xs = _dispatch(xf, tok_sorted, P, h)

    ys = _grouped_ffn(nact, xidx, bmap, xs, W1.astype(jnp.bfloat16),
                      b1[:, None, :], W2.astype(jnp.bfloat16),
                      b2[:, None, :], w_sorted[:, None], NB, P, h, F)

    out = _combine(ys, pos[:T], pos[T:], T, h)
    return out.reshape(b, s, h)


# speedup vs baseline: 1.3641x; 1.3641x over previous
import functools

import jax
import jax.numpy as jnp
from jax import lax
from jax.experimental import pallas as pl
from jax.experimental.pallas import tpu as pltpu
from jax.experimental.pallas import tpu_sc as plsc

_E = 16
_TOPK = 2
_BM = 256
_NC = 2
_NS = 16
_NW = _NC * _NS
_L = 16


def _router_body(x_ref, wr_ref, br_ref, pos_ref, wk_ref, bmap_ref, xidx_ref,
                 nact_ref, NB):
    T, E = x_ref.shape[0], wr_ref.shape[1]
    logits = jnp.dot(x_ref[...], wr_ref[...],
                     preferred_element_type=jnp.float32) + br_ref[...]
    m = jnp.max(logits, axis=1, keepdims=True)
    p = jnp.exp(logits - m)
    Z = jnp.sum(p, axis=1, keepdims=True)
    ii = lax.broadcasted_iota(jnp.int32, (T, E), 1)
    e0 = jnp.min(jnp.where(logits == m, ii, E), axis=1, keepdims=True)
    oh0 = ii == e0
    logits1 = jnp.where(oh0, jnp.float32(-1e30), logits)
    l1 = jnp.max(logits1, axis=1, keepdims=True)
    e1 = jnp.min(jnp.where(logits1 == l1, ii, E), axis=1, keepdims=True)
    oh1 = ii == e1
    w0 = 1.0 / Z
    w1 = jnp.exp(l1 - m) / Z

    oh = jnp.concatenate([oh0, oh1], axis=0).astype(jnp.int32)
    inc = oh
    k = 1
    while k < 2 * T:
        inc = inc + jnp.concatenate(
            [jnp.zeros((k, E), jnp.int32), inc[:-k]], axis=0)
        k *= 2
    excl = inc - oh
    rank = jnp.sum(excl * oh, axis=1, keepdims=True)
    counts = jnp.sum(oh, axis=0, keepdims=True)
    blocks = (counts + (_BM - 1)) // _BM
    binc = blocks
    k = 1
    while k < E:
        binc = binc + jnp.concatenate(
            [jnp.zeros((1, k), jnp.int32), binc[:, :-k]], axis=1)
        k *= 2
    bstart = binc - blocks
    seg = _BM * bstart
    pos = jnp.sum(seg * oh, axis=1, keepdims=True) + rank
    nact = jnp.sum(blocks)

    g = lax.broadcasted_iota(jnp.int32, (1, NB), 1)
    acc = jnp.zeros((1, NB), jnp.int32)
    for e in range(E):
        acc = acc + (g >= bstart[0, e]).astype(jnp.int32)
    bmap_raw = acc - 1
    last = jnp.sum(jnp.where(g == nact - 1, bmap_raw, 0), axis=1,
                   keepdims=True)
    pos_ref[...] = pos
    wk_ref[...] = jnp.concatenate([w0, w1], axis=0)
    bmap_ref[...] = jnp.where(g < nact, bmap_raw, last)
    xidx_ref[...] = jnp.where(g < nact, g, nact - 1)
    nact_ref[...] = jnp.full((1, 1), 0, jnp.int32) + nact


def _router(xf, Wr, br, NB):
    T = xf.shape[0]
    return pl.pallas_call(
        functools.partial(_router_body, NB=NB),
        out_shape=[
            jax.ShapeDtypeStruct((_TOPK * T, 1), jnp.int32),
            jax.ShapeDtypeStruct((_TOPK * T, 1), jnp.float32),
            jax.ShapeDtypeStruct((1, NB), jnp.int32),
            jax.ShapeDtypeStruct((1, NB), jnp.int32),
            jax.ShapeDtypeStruct((1, 1), jnp.int32),
        ],
    )(xf, Wr, br[None, :])


def _ffn_body(nact_ref, xidx_ref, bmap_ref, x_ref, w1_ref, b1_ref, w2_ref,
              b2_ref, wcol_ref, o_ref):
    g = pl.program_id(0)

    @pl.when(g < nact_ref[0])
    def _():
        hmid = jnp.dot(x_ref[...], w1_ref[0],
                       preferred_element_type=jnp.float32)
        hmid = jnp.maximum(hmid + b1_ref[0], 0.0)
        y = jnp.dot(hmid, w2_ref[0],
                    preferred_element_type=jnp.float32)
        y = y + b2_ref[0]
        o_ref[...] = y * wcol_ref[...]


def _grouped_ffn(nact, xidx, bmap, xs, W1, b1, W2, b2, w_col, NB, P, H, F):
    grid_spec = pltpu.PrefetchScalarGridSpec(
        num_scalar_prefetch=3,
        grid=(NB,),
        in_specs=[
            pl.BlockSpec((_BM, H), lambda g, n, xi, bm: (xi[g], 0)),
            pl.BlockSpec((1, H, F), lambda g, n, xi, bm: (bm[g], 0, 0)),
            pl.BlockSpec((1, 1, F), lambda g, n, xi, bm: (bm[g], 0, 0)),
            pl.BlockSpec((1, F, H), lambda g, n, xi, bm: (bm[g], 0, 0)),
            pl.BlockSpec((1, 1, H), lambda g, n, xi, bm: (bm[g], 0, 0)),
            pl.BlockSpec((_BM, 1), lambda g, n, xi, bm: (xi[g], 0)),
        ],
        out_specs=pl.BlockSpec((_BM, H), lambda g, n, xi, bm: (xi[g], 0)),
    )
    return pl.pallas_call(
        _ffn_body,
        grid_spec=grid_spec,
        out_shape=jax.ShapeDtypeStruct((P, H), jnp.float32),
    )(nact, xidx, bmap, xs, W1, b1, W2, b2, w_col)


def _dispatch(xf, pos, w_flat, tok_in, tok_base, zeros_f, P, H):
    A = pos.shape[0]
    a_pt = A // _NS
    rows_pw = P // _NW
    CH = 64
    NCH = rows_pw // CH
    mesh = plsc.VectorSubcoreMesh(core_axis_name="c", subcore_axis_name="s")

    @functools.partial(
        pl.kernel, mesh=mesh,
        out_type=[jax.ShapeDtypeStruct((P, H), jnp.float32),
                  jax.ShapeDtypeStruct((P,), jnp.float32)],
        scratch_types=(
            [pltpu.VMEM((a_pt,), jnp.int32),
             pltpu.VMEM((a_pt,), jnp.int32),
             pltpu.VMEM((a_pt,), jnp.float32),
             pltpu.VMEM_SHARED((P,), jnp.int32),
             pltpu.VMEM_SHARED((P,), jnp.float32)] +
            [pltpu.VMEM((CH,), jnp.int32) for _ in range(NCH)] +
            [pltpu.VMEM((CH, H), jnp.float32),
             pltpu.VMEM((CH, H), jnp.float32),
             pltpu.SemaphoreType.DMA,
             pltpu.SemaphoreType.DMA]
        ),
    )
    def k(xf_h, pos_h, w_h, tok_h, tb_h, zf_h, xs_h, ws_h, *rest):
        pos_v, tokv_v, wv_v, sh_tok, sh_w = rest[:5]
        idx = rest[5:5 + NCH]
        buf0, buf1, sem0, sem1 = rest[5 + NCH:]
        s = lax.axis_index("s")
        wid = s * _NC + lax.axis_index("c")

        @pl.when(s == 0)
        def _():
            pltpu.sync_copy(tb_h, sh_tok)
            pltpu.sync_copy(zf_h, sh_w)

        pltpu.sync_copy(pos_h.at[pl.ds(s * a_pt, a_pt)], pos_v)
        pltpu.sync_copy(tok_h.at[pl.ds(s * a_pt, a_pt)], tokv_v)
        pltpu.sync_copy(w_h.at[pl.ds(s * a_pt, a_pt)], wv_v)
        plsc.subcore_barrier()
        pltpu.sync_copy(tokv_v, sh_tok.at[pos_v])
        pltpu.sync_copy(wv_v, sh_w.at[pos_v])
        plsc.subcore_barrier()

        base = wid * rows_pw
        for c in range(NCH):
            pltpu.sync_copy(sh_tok.at[pl.ds(base + c * CH, CH)], idx[c])
        bufs = (buf0, buf1)
        sems = (sem0, sem1)
        copies = [None, None]
        copies[0] = pltpu.async_copy(xf_h.at[idx[0]], buf0, sem0)
        for c in range(NCH):
            if c + 1 < NCH:
                copies[(c + 1) % 2] = pltpu.async_copy(
                    xf_h.at[idx[c + 1]], bufs[(c + 1) % 2],
                    sems[(c + 1) % 2])
            copies[c % 2].wait()
            pltpu.sync_copy(bufs[c % 2], xs_h.at[pl.ds(base + c * CH, CH)])
        pltpu.sync_copy(sh_w.at[pl.ds(base, rows_pw)],
                        ws_h.at[pl.ds(base, rows_pw)])

    return k(xf, pos, w_flat, tok_in, tok_base, zeros_f)


def _combine(ys, p0, p1, T, H):
    tpw = T // _NW
    mesh = plsc.VectorSubcoreMesh(core_axis_name="c", subcore_axis_name="s")

    @functools.partial(
        pl.kernel, mesh=mesh,
        out_type=jax.ShapeDtypeStruct((T, H), jnp.float32),
        scratch_types=[
            pltpu.VMEM((tpw,), jnp.int32),
            pltpu.VMEM((tpw,), jnp.int32),
            pltpu.VMEM((tpw, H), jnp.float32),
            pltpu.VMEM((tpw, H), jnp.float32),
            pltpu.SemaphoreType.DMA,
            pltpu.SemaphoreType.DMA,
        ],
    )
    def k(ys_h, p0_h, p1_h, out_h, i0_v, i1_v, ba, bb, s0, s1):
        wid = lax.axis_index("s") * _NC + lax.axis_index("c")
        base = wid * tpw
        pltpu.sync_copy(p0_h.at[pl.ds(base, tpw)], i0_v)
        pltpu.sync_copy(p1_h.at[pl.ds(base, tpw)], i1_v)
        ca = pltpu.async_copy(ys_h.at[i0_v], ba, s0)
        cb = pltpu.async_copy(ys_h.at[i1_v], bb, s1)
        ca.wait()
        cb.wait()

        def addrow(r, carry):
            for j in range(H // _L):
                sl = pl.ds(j * _L, _L)
                ba[r, sl] = ba[r, sl] + bb[r, sl]
            return carry

        lax.fori_loop(0, tpw, addrow, 0)
        pltpu.sync_copy(ba, out_h.at[pl.ds(base, tpw)])

    return k(ys, p0, p1)


def kernel(x, Wr, br, W1, b1, W2, b2):
    b, s, h = x.shape
    T = b * s
    F = W1.shape[-1]
    E = Wr.shape[-1]
    xf = x.reshape(T, h)

    NB = (_TOPK * T) // _BM + E
    P = NB * _BM

    pos2d, wk2d, bmap2d, xidx2d, nact2d = _router(xf, Wr, br, NB)
    pos = pos2d.reshape(-1)
    w_flat = wk2d.reshape(-1)
    bmap = bmap2d.reshape(-1)
    xidx = xidx2d.reshape(-1)
    nact = nact2d.reshape(-1)

    tok = jnp.tile(jnp.arange(T, dtype=jnp.int32), _TOPK)
    tok_base = jnp.arange(P, dtype=jnp.int32) % T
    zeros_f = jnp.zeros((P,), jnp.float32)

    xs, w_sorted = _dispatch(xf, pos, w_flat, tok, tok_base, zeros_f, P, h)

    ys = _grouped_ffn(nact, xidx, bmap, xs, W1, b1[:, None, :], W2,
                      b2[:, None, :], w_sorted[:, None], NB, P, h, F)

    out = _combine(ys, pos[:T], pos[T:], T, h)
    return out.reshape(b, s, h)


# speedup vs baseline: 1.3859x; 1.0160x over previous
import functools

import jax
import jax.numpy as jnp
from jax import lax
from jax.experimental import pallas as pl
from jax.experimental.pallas import tpu as pltpu
from jax.experimental.pallas import tpu_sc as plsc

_E = 16
_TOPK = 2
_BM = 256
_NC = 2
_NS = 16
_NW = _NC * _NS
_L = 16


def _router_body(x_ref, wr_ref, br_ref, pos_ref, wk_ref, bmap_ref, xidx_ref,
                 nact_ref, NB):
    T, E = x_ref.shape[0], wr_ref.shape[1]
    logits = jnp.dot(x_ref[...], wr_ref[...],
                     preferred_element_type=jnp.float32) + br_ref[...]
    m = jnp.max(logits, axis=1, keepdims=True)
    p = jnp.exp(logits - m)
    Z = jnp.sum(p, axis=1, keepdims=True)
    ii = lax.broadcasted_iota(jnp.int32, (T, E), 1)
    e0 = jnp.min(jnp.where(logits == m, ii, E), axis=1, keepdims=True)
    oh0 = ii == e0
    logits1 = jnp.where(oh0, jnp.float32(-1e30), logits)
    l1 = jnp.max(logits1, axis=1, keepdims=True)
    e1 = jnp.min(jnp.where(logits1 == l1, ii, E), axis=1, keepdims=True)
    oh1 = ii == e1
    w0 = 1.0 / Z
    w1 = jnp.exp(l1 - m) / Z

    oh = jnp.concatenate([oh0, oh1], axis=0).astype(jnp.int32)
    inc = oh
    k = 1
    while k < 2 * T:
        inc = inc + jnp.concatenate(
            [jnp.zeros((k, E), jnp.int32), inc[:-k]], axis=0)
        k *= 2
    excl = inc - oh
    rank = jnp.sum(excl * oh, axis=1, keepdims=True)
    counts = jnp.sum(oh, axis=0, keepdims=True)
    blocks = (counts + (_BM - 1)) // _BM
    binc = blocks
    k = 1
    while k < E:
        binc = binc + jnp.concatenate(
            [jnp.zeros((1, k), jnp.int32), binc[:, :-k]], axis=1)
        k *= 2
    bstart = binc - blocks
    seg = _BM * bstart
    pos = jnp.sum(seg * oh, axis=1, keepdims=True) + rank
    nact = jnp.sum(blocks)

    g = lax.broadcasted_iota(jnp.int32, (1, NB), 1)
    acc = jnp.zeros((1, NB), jnp.int32)
    for e in range(E):
        acc = acc + (g >= bstart[0, e]).astype(jnp.int32)
    bmap_raw = acc - 1
    last = jnp.sum(jnp.where(g == nact - 1, bmap_raw, 0), axis=1,
                   keepdims=True)
    pos_ref[...] = pos
    wk_ref[...] = jnp.concatenate([w0, w1], axis=0)
    bmap_ref[...] = jnp.where(g < nact, bmap_raw, last)
    xidx_ref[...] = jnp.where(g < nact, g, nact - 1)
    nact_ref[...] = jnp.full((1, 1), 0, jnp.int32) + nact


def _router(xf, Wr, br, NB):
    T = xf.shape[0]
    return pl.pallas_call(
        functools.partial(_router_body, NB=NB),
        out_shape=[
            jax.ShapeDtypeStruct((_TOPK * T, 1), jnp.int32),
            jax.ShapeDtypeStruct((_TOPK * T, 1), jnp.float32),
            jax.ShapeDtypeStruct((1, NB), jnp.int32),
            jax.ShapeDtypeStruct((1, NB), jnp.int32),
            jax.ShapeDtypeStruct((1, 1), jnp.int32),
        ],
    )(xf, Wr, br[None, :])


def _ffn_body(nact_ref, xidx_ref, bmap_ref, x_ref, w1_ref, b1_ref, w2_ref,
              b2_ref, wcol_ref, o_ref):
    g = pl.program_id(0)

    @pl.when(g < nact_ref[0])
    def _():
        hmid = jnp.dot(x_ref[...], w1_ref[0],
                       preferred_element_type=jnp.float32)
        hmid = jnp.maximum(hmid + b1_ref[0], 0.0)
        y = jnp.dot(hmid, w2_ref[0],
                    preferred_element_type=jnp.float32)
        y = y + b2_ref[0]
        o_ref[...] = y * wcol_ref[...]


def _grouped_ffn(nact, xidx, bmap, xs, W1, b1, W2, b2, w_col, NB, P, H, F):
    grid_spec = pltpu.PrefetchScalarGridSpec(
        num_scalar_prefetch=3,
        grid=(NB,),
        in_specs=[
            pl.BlockSpec((_BM, H), lambda g, n, xi, bm: (xi[g], 0)),
            pl.BlockSpec((1, H, F), lambda g, n, xi, bm: (bm[g], 0, 0)),
            pl.BlockSpec((1, 1, F), lambda g, n, xi, bm: (bm[g], 0, 0)),
            pl.BlockSpec((1, F, H), lambda g, n, xi, bm: (bm[g], 0, 0)),
            pl.BlockSpec((1, 1, H), lambda g, n, xi, bm: (bm[g], 0, 0)),
            pl.BlockSpec((_BM, 1), lambda g, n, xi, bm: (xi[g], 0)),
        ],
        out_specs=pl.BlockSpec((_BM, H), lambda g, n, xi, bm: (xi[g], 0)),
    )
    return pl.pallas_call(
        _ffn_body,
        grid_spec=grid_spec,
        out_shape=jax.ShapeDtypeStruct((P, H), jnp.float32),
    )(nact, xidx, bmap, xs, W1, b1, W2, b2, w_col)


def _dispatch(xf, pos, w_flat, tok_base, P, H):
    A = pos.shape[0]
    a_pt = A // _NS
    rows_pw = P // _NW
    CH = 64
    NCH = rows_pw // CH
    mesh = plsc.VectorSubcoreMesh(core_axis_name="c", subcore_axis_name="s")

    @functools.partial(
        pl.kernel, mesh=mesh,
        out_type=[jax.ShapeDtypeStruct((P, H), jnp.float32),
                  jax.ShapeDtypeStruct((P,), jnp.float32)],
        scratch_types=(
            [pltpu.VMEM((a_pt,), jnp.int32),
             pltpu.VMEM((a_pt,), jnp.int32),
             pltpu.VMEM((a_pt,), jnp.float32),
             pltpu.VMEM_SHARED((P,), jnp.int32),
             pltpu.VMEM_SHARED((P,), jnp.float32)] +
            [pltpu.VMEM((CH,), jnp.int32) for _ in range(NCH)] +
            [pltpu.VMEM((CH, H), jnp.float32),
             pltpu.VMEM((CH, H), jnp.float32),
             pltpu.SemaphoreType.DMA,
             pltpu.SemaphoreType.DMA]
        ),
    )
    def k(xf_h, pos_h, w_h, tb_h, xs_h, ws_h, *rest):
        pos_v, tokv_v, wv_v, sh_tok, sh_w = rest[:5]
        idx = rest[5:5 + NCH]
        buf0, buf1, sem0, sem1 = rest[5 + NCH:]
        s = lax.axis_index("s")
        wid = s * _NC + lax.axis_index("c")

        @pl.when(s == 0)
        def _():
            pltpu.sync_copy(tb_h, sh_tok)

        pltpu.sync_copy(pos_h.at[pl.ds(s * a_pt, a_pt)], pos_v)
        pltpu.sync_copy(w_h.at[pl.ds(s * a_pt, a_pt)], wv_v)
        T = xf_h.shape[0]

        def _tokfill(i, carry):
            tokv_v[pl.ds(i * _L, _L)] = lax.rem(
                s * a_pt + i * _L + lax.iota(jnp.int32, _L), T)
            return carry

        lax.fori_loop(0, a_pt // _L, _tokfill, 0)
        plsc.subcore_barrier()
        pltpu.sync_copy(tokv_v, sh_tok.at[pos_v])
        pltpu.sync_copy(wv_v, sh_w.at[pos_v])
        plsc.subcore_barrier()

        base = wid * rows_pw
        for c in range(NCH):
            pltpu.sync_copy(sh_tok.at[pl.ds(base + c * CH, CH)], idx[c])
        bufs = (buf0, buf1)
        sems = (sem0, sem1)
        copies = [None, None]
        copies[0] = pltpu.async_copy(xf_h.at[idx[0]], buf0, sem0)
        for c in range(NCH):
            if c + 1 < NCH:
                copies[(c + 1) % 2] = pltpu.async_copy(
                    xf_h.at[idx[c + 1]], bufs[(c + 1) % 2],
                    sems[(c + 1) % 2])
            copies[c % 2].wait()
            pltpu.sync_copy(bufs[c % 2], xs_h.at[pl.ds(base + c * CH, CH)])
        pltpu.sync_copy(sh_w.at[pl.ds(base, rows_pw)],
                        ws_h.at[pl.ds(base, rows_pw)])

    return k(xf, pos, w_flat, tok_base)


def _combine(ys, pos, T, H):
    tpw = T // _NW
    mesh = plsc.VectorSubcoreMesh(core_axis_name="c", subcore_axis_name="s")

    @functools.partial(
        pl.kernel, mesh=mesh,
        out_type=jax.ShapeDtypeStruct((T, H), jnp.float32),
        scratch_types=[
            pltpu.VMEM((tpw,), jnp.int32),
            pltpu.VMEM((tpw,), jnp.int32),
            pltpu.VMEM((tpw, H), jnp.float32),
            pltpu.VMEM((tpw, H), jnp.float32),
            pltpu.SemaphoreType.DMA,
            pltpu.SemaphoreType.DMA,
        ],
    )
    def k(ys_h, pos_h, out_h, i0_v, i1_v, ba, bb, s0, s1):
        wid = lax.axis_index("s") * _NC + lax.axis_index("c")
        base = wid * tpw
        pltpu.sync_copy(pos_h.at[pl.ds(base, tpw)], i0_v)
        pltpu.sync_copy(pos_h.at[pl.ds(T + base, tpw)], i1_v)
        ca = pltpu.async_copy(ys_h.at[i0_v], ba, s0)
        cb = pltpu.async_copy(ys_h.at[i1_v], bb, s1)
        ca.wait()
        cb.wait()

        def addrow(r, carry):
            for j in range(H // _L):
                sl = pl.ds(j * _L, _L)
                ba[r, sl] = ba[r, sl] + bb[r, sl]
            return carry

        lax.fori_loop(0, tpw, addrow, 0)
        pltpu.sync_copy(ba, out_h.at[pl.ds(base, tpw)])

    return k(ys, pos)


def kernel(x, Wr, br, W1, b1, W2, b2):
    b, s, h = x.shape
    T = b * s
    F = W1.shape[-1]
    E = Wr.shape[-1]
    xf = x.reshape(T, h)

    NB = (_TOPK * T) // _BM + E
    P = NB * _BM

    pos2d, wk2d, bmap2d, xidx2d, nact2d = _router(xf, Wr, br, NB)
    pos = pos2d.reshape(-1)
    w_flat = wk2d.reshape(-1)
    bmap = bmap2d.reshape(-1)
    xidx = xidx2d.reshape(-1)
    nact = nact2d.reshape(-1)

    tok_base = jnp.arange(P, dtype=jnp.int32) % T

    xs, w_sorted = _dispatch(xf, pos, w_flat, tok_base, P, h)

    ys = _grouped_ffn(nact, xidx, bmap, xs, W1, b1[:, None, :], W2,
                      b2[:, None, :], w_sorted[:, None], NB, P, h, F)

    out = _combine(ys, pos, T, h)
    return out.reshape(b, s, h)


# speedup vs baseline: 1.4238x; 1.0273x over previous
import functools

import jax
import jax.numpy as jnp
from jax import lax
from jax.experimental import pallas as pl
from jax.experimental.pallas import tpu as pltpu
from jax.experimental.pallas import tpu_sc as plsc

_E = 16
_TOPK = 2
_BM = 256
_NC = 2
_NS = 16
_NW = _NC * _NS
_L = 16


def _router_body(x_ref, wr_ref, br_ref, pos_ref, wk_ref, bmap_ref, xidx_ref,
                 nact_ref, NB):
    T, E = x_ref.shape[0], wr_ref.shape[1]
    logits = jnp.dot(x_ref[...], wr_ref[...],
                     preferred_element_type=jnp.float32) + br_ref[...]
    m = jnp.max(logits, axis=1, keepdims=True)
    p = jnp.exp(logits - m)
    Z = jnp.sum(p, axis=1, keepdims=True)
    ii = lax.broadcasted_iota(jnp.int32, (T, E), 1)
    e0 = jnp.min(jnp.where(logits == m, ii, E), axis=1, keepdims=True)
    oh0 = ii == e0
    logits1 = jnp.where(oh0, jnp.float32(-1e30), logits)
    l1 = jnp.max(logits1, axis=1, keepdims=True)
    e1 = jnp.min(jnp.where(logits1 == l1, ii, E), axis=1, keepdims=True)
    oh1 = ii == e1
    w0 = 1.0 / Z
    w1 = jnp.exp(l1 - m) / Z

    oh = jnp.concatenate([oh0, oh1], axis=0).astype(jnp.int32)
    inc = oh
    k = 1
    while k < 2 * T:
        inc = inc + jnp.concatenate(
            [jnp.zeros((k, E), jnp.int32), inc[:-k]], axis=0)
        k *= 2
    excl = inc - oh
    rank = jnp.sum(excl * oh, axis=1, keepdims=True)
    counts = jnp.sum(oh, axis=0, keepdims=True)
    blocks = (counts + (_BM - 1)) // _BM
    binc = blocks
    k = 1
    while k < E:
        binc = binc + jnp.concatenate(
            [jnp.zeros((1, k), jnp.int32), binc[:, :-k]], axis=1)
        k *= 2
    bstart = binc - blocks
    seg = _BM * bstart
    pos = jnp.sum(seg * oh, axis=1, keepdims=True) + rank
    nact = jnp.sum(blocks)

    g = lax.broadcasted_iota(jnp.int32, (1, NB), 1)
    acc = jnp.zeros((1, NB), jnp.int32)
    for e in range(E):
        acc = acc + (g >= bstart[0, e]).astype(jnp.int32)
    bmap_raw = acc - 1
    last = jnp.sum(jnp.where(g == nact - 1, bmap_raw, 0), axis=1,
                   keepdims=True)
    pos_ref[...] = jnp.reshape(pos, (1, _TOPK * T))
    wk_ref[...] = jnp.reshape(jnp.concatenate([w0, w1], axis=0),
                              (1, _TOPK * T))
    bmap_ref[...] = jnp.where(g < nact, bmap_raw, last)
    xidx_ref[...] = jnp.where(g < nact, g, nact - 1)
    nact_ref[...] = jnp.full((1, 1), 0, jnp.int32) + nact


def _router(xf, Wr, br, NB):
    T = xf.shape[0]
    return pl.pallas_call(
        functools.partial(_router_body, NB=NB),
        out_shape=[
            jax.ShapeDtypeStruct((1, _TOPK * T), jnp.int32),
            jax.ShapeDtypeStruct((1, _TOPK * T), jnp.float32),
            jax.ShapeDtypeStruct((1, NB), jnp.int32),
            jax.ShapeDtypeStruct((1, NB), jnp.int32),
            jax.ShapeDtypeStruct((1, 1), jnp.int32),
        ],
    )(xf, Wr, br[None, :])


def _ffn_body(nact_ref, xidx_ref, bmap_ref, x_ref, w1_ref, b1_ref, w2_ref,
              b2_ref, wcol_ref, o_ref):
    g = pl.program_id(0)

    @pl.when(g < nact_ref[0])
    def _():
        hmid = jnp.dot(x_ref[...], w1_ref[0],
                       preferred_element_type=jnp.float32)
        hmid = jnp.maximum(hmid + b1_ref[0], 0.0)
        y = jnp.dot(hmid, w2_ref[0],
                    preferred_element_type=jnp.float32)
        y = y + b2_ref[0]
        o_ref[...] = y * wcol_ref[...]


def _grouped_ffn(nact, xidx, bmap, xs, W1, b1, W2, b2, w_col, NB, P, H, F):
    grid_spec = pltpu.PrefetchScalarGridSpec(
        num_scalar_prefetch=3,
        grid=(NB,),
        in_specs=[
            pl.BlockSpec((_BM, H), lambda g, n, xi, bm: (xi[g], 0)),
            pl.BlockSpec((1, H, F), lambda g, n, xi, bm: (bm[g], 0, 0)),
            pl.BlockSpec((1, 1, F), lambda g, n, xi, bm: (bm[g], 0, 0)),
            pl.BlockSpec((1, F, H), lambda g, n, xi, bm: (bm[g], 0, 0)),
            pl.BlockSpec((1, 1, H), lambda g, n, xi, bm: (bm[g], 0, 0)),
            pl.BlockSpec((_BM, 1), lambda g, n, xi, bm: (xi[g], 0)),
        ],
        out_specs=pl.BlockSpec((_BM, H), lambda g, n, xi, bm: (xi[g], 0)),
    )
    return pl.pallas_call(
        _ffn_body,
        grid_spec=grid_spec,
        out_shape=jax.ShapeDtypeStruct((P, H), jnp.float32),
    )(nact, xidx, bmap, xs, W1, b1, W2, b2, w_col)


def _dispatch(xf, pos, w_flat, tok_base, P, H):
    A = pos.shape[0]
    a_pt = A // _NS
    rows_pw = P // _NW
    CH = 64
    NCH = rows_pw // CH
    mesh = plsc.VectorSubcoreMesh(core_axis_name="c", subcore_axis_name="s")

    @functools.partial(
        pl.kernel, mesh=mesh,
        out_type=[jax.ShapeDtypeStruct((P, H), jnp.float32),
                  jax.ShapeDtypeStruct((P,), jnp.float32)],
        scratch_types=(
            [pltpu.VMEM((a_pt,), jnp.int32),
             pltpu.VMEM((a_pt,), jnp.int32),
             pltpu.VMEM((a_pt,), jnp.float32),
             pltpu.VMEM_SHARED((P,), jnp.int32),
             pltpu.VMEM_SHARED((P,), jnp.float32)] +
            [pltpu.VMEM((CH,), jnp.int32) for _ in range(NCH)] +
            [pltpu.VMEM((CH, H), jnp.float32),
             pltpu.VMEM((CH, H), jnp.float32),
             pltpu.SemaphoreType.DMA,
             pltpu.SemaphoreType.DMA]
        ),
    )
    def k(xf_h, pos_h, w_h, tb_h, xs_h, ws_h, *rest):
        pos_v, tokv_v, wv_v, sh_tok, sh_w = rest[:5]
        idx = rest[5:5 + NCH]
        buf0, buf1, sem0, sem1 = rest[5 + NCH:]
        s = lax.axis_index("s")
        wid = s * _NC + lax.axis_index("c")

        @pl.when(s == 0)
        def _():
            pltpu.sync_copy(tb_h, sh_tok)

        pltpu.sync_copy(pos_h.at[pl.ds(s * a_pt, a_pt)], pos_v)
        pltpu.sync_copy(w_h.at[pl.ds(s * a_pt, a_pt)], wv_v)
        T = xf_h.shape[0]

        def _tokfill(i, carry):
            tokv_v[pl.ds(i * _L, _L)] = lax.rem(
                s * a_pt + i * _L + lax.iota(jnp.int32, _L), T)
            return carry

        lax.fori_loop(0, a_pt // _L, _tokfill, 0)
        plsc.subcore_barrier()
        pltpu.sync_copy(tokv_v, sh_tok.at[pos_v])
        pltpu.sync_copy(wv_v, sh_w.at[pos_v])
        plsc.subcore_barrier()

        base = wid * rows_pw
        for c in range(NCH):
            pltpu.sync_copy(sh_tok.at[pl.ds(base + c * CH, CH)], idx[c])
        bufs = (buf0, buf1)
        sems = (sem0, sem1)
        copies = [None, None]
        copies[0] = pltpu.async_copy(xf_h.at[idx[0]], buf0, sem0)
        for c in range(NCH):
            if c + 1 < NCH:
                copies[(c + 1) % 2] = pltpu.async_copy(
                    xf_h.at[idx[c + 1]], bufs[(c + 1) % 2],
                    sems[(c + 1) % 2])
            copies[c % 2].wait()
            pltpu.sync_copy(bufs[c % 2], xs_h.at[pl.ds(base + c * CH, CH)])
        pltpu.sync_copy(sh_w.at[pl.ds(base, rows_pw)],
                        ws_h.at[pl.ds(base, rows_pw)])

    return k(xf, pos, w_flat, tok_base)


def _combine(ys, pos, T, H):
    tpw = T // _NW
    mesh = plsc.VectorSubcoreMesh(core_axis_name="c", subcore_axis_name="s")

    @functools.partial(
        pl.kernel, mesh=mesh,
        out_type=jax.ShapeDtypeStruct((T, H), jnp.float32),
        scratch_types=[
            pltpu.VMEM((tpw,), jnp.int32),
            pltpu.VMEM((tpw,), jnp.int32),
            pltpu.VMEM((tpw, H), jnp.float32),
            pltpu.VMEM((tpw, H), jnp.float32),
            pltpu.SemaphoreType.DMA,
            pltpu.SemaphoreType.DMA,
        ],
    )
    def k(ys_h, pos_h, out_h, i0_v, i1_v, ba, bb, s0, s1):
        wid = lax.axis_index("s") * _NC + lax.axis_index("c")
        base = wid * tpw
        pltpu.sync_copy(pos_h.at[pl.ds(base, tpw)], i0_v)
        pltpu.sync_copy(pos_h.at[pl.ds(T + base, tpw)], i1_v)
        ca = pltpu.async_copy(ys_h.at[i0_v], ba, s0)
        cb = pltpu.async_copy(ys_h.at[i1_v], bb, s1)
        ca.wait()
        cb.wait()

        def addrow(r, carry):
            for j in range(H // _L):
                sl = pl.ds(j * _L, _L)
                ba[r, sl] = ba[r, sl] + bb[r, sl]
            return carry

        lax.fori_loop(0, tpw, addrow, 0)
        pltpu.sync_copy(ba, out_h.at[pl.ds(base, tpw)])

    return k(ys, pos)


def kernel(x, Wr, br, W1, b1, W2, b2):
    b, s, h = x.shape
    T = b * s
    F = W1.shape[-1]
    E = Wr.shape[-1]
    xf = x.reshape(T, h)

    NB = (_TOPK * T) // _BM + E
    P = NB * _BM

    pos2d, wk2d, bmap2d, xidx2d, nact2d = _router(xf, Wr, br, NB)
    pos = pos2d.reshape(-1)
    w_flat = wk2d.reshape(-1)
    bmap = bmap2d.reshape(-1)
    xidx = xidx2d.reshape(-1)
    nact = nact2d.reshape(-1)

    tok_base = jnp.arange(P, dtype=jnp.int32) % T

    xs, w_sorted = _dispatch(xf, pos, w_flat, tok_base, P, h)

    ys = _grouped_ffn(nact, xidx, bmap, xs, W1, b1[:, None, :], W2,
                      b2[:, None, :], w_sorted[:, None], NB, P, h, F)

    out = _combine(ys, pos, T, h)
    return out.reshape(b, s, h)


# speedup vs baseline: 1.4263x; 1.0017x over previous
import functools

import jax
import jax.numpy as jnp
from jax import lax
from jax.experimental import pallas as pl
from jax.experimental.pallas import tpu as pltpu
from jax.experimental.pallas import tpu_sc as plsc

_E = 16
_TOPK = 2
_BM = 256
_NC = 2
_NS = 16
_NW = _NC * _NS
_L = 16


def _router_body(x_ref, wr_ref, br_ref, pos_ref, wk_ref, bmap_ref, xidx_ref,
                 nact_ref, NB):
    T, E = x_ref.shape[0], wr_ref.shape[1]
    logits = jnp.dot(x_ref[...], wr_ref[...],
                     preferred_element_type=jnp.float32) + br_ref[...]
    m = jnp.max(logits, axis=1, keepdims=True)
    p = jnp.exp(logits - m)
    Z = jnp.sum(p, axis=1, keepdims=True)
    ii = lax.broadcasted_iota(jnp.int32, (T, E), 1)
    e0 = jnp.min(jnp.where(logits == m, ii, E), axis=1, keepdims=True)
    oh0 = ii == e0
    logits1 = jnp.where(oh0, jnp.float32(-1e30), logits)
    l1 = jnp.max(logits1, axis=1, keepdims=True)
    e1 = jnp.min(jnp.where(logits1 == l1, ii, E), axis=1, keepdims=True)
    oh1 = ii == e1
    w0 = 1.0 / Z
    w1 = jnp.exp(l1 - m) / Z

    ohf = jnp.concatenate([oh0, oh1], axis=0).astype(jnp.float32)
    R = 512
    rr = lax.broadcasted_iota(jnp.int32, (R, R), 0)
    cc = lax.broadcasted_iota(jnp.int32, (R, R), 1)
    tril = (rr > cc).astype(jnp.float32)
    parts = []
    csum = jnp.zeros((1, E), jnp.float32)
    for i in range((_TOPK * T) // R):
        blk = ohf[i * R:(i + 1) * R]
        parts.append(jnp.dot(tril, blk, preferred_element_type=jnp.float32)
                     + csum)
        csum = csum + jnp.sum(blk, axis=0, keepdims=True)
    excl_f = jnp.concatenate(parts, axis=0)
    oh = ohf.astype(jnp.int32)
    rank = jnp.sum(excl_f * ohf, axis=1,
                   keepdims=True).astype(jnp.int32)
    counts = csum.astype(jnp.int32)
    blocks = (counts + (_BM - 1)) // _BM
    binc = blocks
    k = 1
    while k < E:
        binc = binc + jnp.concatenate(
            [jnp.zeros((1, k), jnp.int32), binc[:, :-k]], axis=1)
        k *= 2
    bstart = binc - blocks
    seg = _BM * bstart
    pos = jnp.sum(seg * oh, axis=1, keepdims=True) + rank
    nact = jnp.sum(blocks)

    g = lax.broadcasted_iota(jnp.int32, (1, NB), 1)
    acc = jnp.zeros((1, NB), jnp.int32)
    for e in range(E):
        acc = acc + (g >= bstart[0, e]).astype(jnp.int32)
    bmap_raw = acc - 1
    last = jnp.sum(jnp.where(g == nact - 1, bmap_raw, 0), axis=1,
                   keepdims=True)
    pos_ref[...] = jnp.reshape(pos, (1, _TOPK * T))
    wk_ref[...] = jnp.reshape(jnp.concatenate([w0, w1], axis=0),
                              (1, _TOPK * T))
    bmap_ref[...] = jnp.where(g < nact, bmap_raw, last)
    xidx_ref[...] = jnp.where(g < nact, g, nact - 1)
    nact_ref[...] = jnp.full((1, 1), 0, jnp.int32) + nact


def _router(xf, Wr, br, NB):
    T = xf.shape[0]
    return pl.pallas_call(
        functools.partial(_router_body, NB=NB),
        out_shape=[
            jax.ShapeDtypeStruct((1, _TOPK * T), jnp.int32),
            jax.ShapeDtypeStruct((1, _TOPK * T), jnp.float32),
            jax.ShapeDtypeStruct((1, NB), jnp.int32),
            jax.ShapeDtypeStruct((1, NB), jnp.int32),
            jax.ShapeDtypeStruct((1, 1), jnp.int32),
        ],
    )(xf, Wr, br[None, :])


def _ffn_body(nact_ref, xidx_ref, bmap_ref, x_ref, w1_ref, b1_ref, w2_ref,
              b2_ref, wcol_ref, o_ref):
    g = pl.program_id(0)

    @pl.when(g < nact_ref[0])
    def _():
        hmid = jnp.dot(x_ref[...], w1_ref[0],
                       preferred_element_type=jnp.float32)
        hmid = jnp.maximum(hmid + b1_ref[0], 0.0)
        y = jnp.dot(hmid, w2_ref[0],
                    preferred_element_type=jnp.float32)
        y = y + b2_ref[0]
        o_ref[...] = y * wcol_ref[...]


def _grouped_ffn(nact, xidx, bmap, xs, W1, b1, W2, b2, w_col, NB, P, H, F):
    grid_spec = pltpu.PrefetchScalarGridSpec(
        num_scalar_prefetch=3,
        grid=(NB,),
        in_specs=[
            pl.BlockSpec((_BM, H), lambda g, n, xi, bm: (xi[g], 0)),
            pl.BlockSpec((1, H, F), lambda g, n, xi, bm: (bm[g], 0, 0)),
            pl.BlockSpec((1, 1, F), lambda g, n, xi, bm: (bm[g], 0, 0)),
            pl.BlockSpec((1, F, H), lambda g, n, xi, bm: (bm[g], 0, 0)),
            pl.BlockSpec((1, 1, H), lambda g, n, xi, bm: (bm[g], 0, 0)),
            pl.BlockSpec((_BM, 1), lambda g, n, xi, bm: (xi[g], 0)),
        ],
        out_specs=pl.BlockSpec((_BM, H), lambda g, n, xi, bm: (xi[g], 0)),
    )
    return pl.pallas_call(
        _ffn_body,
        grid_spec=grid_spec,
        out_shape=jax.ShapeDtypeStruct((P, H), jnp.float32),
    )(nact, xidx, bmap, xs, W1, b1, W2, b2, w_col)


def _dispatch(xf, pos, w_flat, tok_base, P, H):
    A = pos.shape[0]
    a_pt = A // _NS
    rows_pw = P // _NW
    CH = 64
    NCH = rows_pw // CH
    mesh = plsc.VectorSubcoreMesh(core_axis_name="c", subcore_axis_name="s")

    @functools.partial(
        pl.kernel, mesh=mesh,
        out_type=[jax.ShapeDtypeStruct((P, H), jnp.float32),
                  jax.ShapeDtypeStruct((P,), jnp.float32)],
        scratch_types=(
            [pltpu.VMEM((a_pt,), jnp.int32),
             pltpu.VMEM((a_pt,), jnp.int32),
             pltpu.VMEM((a_pt,), jnp.float32),
             pltpu.VMEM_SHARED((P,), jnp.int32),
             pltpu.VMEM_SHARED((P,), jnp.float32)] +
            [pltpu.VMEM((CH,), jnp.int32) for _ in range(NCH)] +
            [pltpu.VMEM((CH, H), jnp.float32),
             pltpu.VMEM((CH, H), jnp.float32),
             pltpu.SemaphoreType.DMA,
             pltpu.SemaphoreType.DMA]
        ),
    )
    def k(xf_h, pos_h, w_h, tb_h, xs_h, ws_h, *rest):
        pos_v, tokv_v, wv_v, sh_tok, sh_w = rest[:5]
        idx = rest[5:5 + NCH]
        buf0, buf1, sem0, sem1 = rest[5 + NCH:]
        s = lax.axis_index("s")
        wid = s * _NC + lax.axis_index("c")

        @pl.when(s == 0)
        def _():
            pltpu.sync_copy(tb_h, sh_tok)

        pltpu.sync_copy(pos_h.at[pl.ds(s * a_pt, a_pt)], pos_v)
        pltpu.sync_copy(w_h.at[pl.ds(s * a_pt, a_pt)], wv_v)
        T = xf_h.shape[0]

        def _tokfill(i, carry):
            tokv_v[pl.ds(i * _L, _L)] = lax.rem(
                s * a_pt + i * _L + lax.iota(jnp.int32, _L), T)
            return carry

        lax.fori_loop(0, a_pt // _L, _tokfill, 0)
        plsc.subcore_barrier()
        pltpu.sync_copy(tokv_v, sh_tok.at[pos_v])
        pltpu.sync_copy(wv_v, sh_w.at[pos_v])
        plsc.subcore_barrier()

        base = wid * rows_pw
        for c in range(NCH):
            pltpu.sync_copy(sh_tok.at[pl.ds(base + c * CH, CH)], idx[c])
        bufs = (buf0, buf1)
        sems = (sem0, sem1)
        copies = [None, None]
        copies[0] = pltpu.async_copy(xf_h.at[idx[0]], buf0, sem0)
        for c in range(NCH):
            if c + 1 < NCH:
                copies[(c + 1) % 2] = pltpu.async_copy(
                    xf_h.at[idx[c + 1]], bufs[(c + 1) % 2],
                    sems[(c + 1) % 2])
            copies[c % 2].wait()
            pltpu.sync_copy(bufs[c % 2], xs_h.at[pl.ds(base + c * CH, CH)])
        pltpu.sync_copy(sh_w.at[pl.ds(base, rows_pw)],
                        ws_h.at[pl.ds(base, rows_pw)])

    return k(xf, pos, w_flat, tok_base)


def _combine(ys, pos, T, H):
    tpw = T // _NW
    mesh = plsc.VectorSubcoreMesh(core_axis_name="c", subcore_axis_name="s")

    @functools.partial(
        pl.kernel, mesh=mesh,
        out_type=jax.ShapeDtypeStruct((T, H), jnp.float32),
        scratch_types=[
            pltpu.VMEM((tpw,), jnp.int32),
            pltpu.VMEM((tpw,), jnp.int32),
            pltpu.VMEM((tpw, H), jnp.float32),
            pltpu.VMEM((tpw, H), jnp.float32),
            pltpu.SemaphoreType.DMA,
            pltpu.SemaphoreType.DMA,
        ],
    )
    def k(ys_h, pos_h, out_h, i0_v, i1_v, ba, bb, s0, s1):
        wid = lax.axis_index("s") * _NC + lax.axis_index("c")
        base = wid * tpw
        pltpu.sync_copy(pos_h.at[pl.ds(base, tpw)], i0_v)
        pltpu.sync_copy(pos_h.at[pl.ds(T + base, tpw)], i1_v)
        ca = pltpu.async_copy(ys_h.at[i0_v], ba, s0)
        cb = pltpu.async_copy(ys_h.at[i1_v], bb, s1)
        ca.wait()
        cb.wait()

        def addrow(r, carry):
            for j in range(H // _L):
                sl = pl.ds(j * _L, _L)
                ba[r, sl] = ba[r, sl] + bb[r, sl]
            return carry

        lax.fori_loop(0, tpw, addrow, 0)
        pltpu.sync_copy(ba, out_h.at[pl.ds(base, tpw)])

    return k(ys, pos)


def kernel(x, Wr, br, W1, b1, W2, b2):
    b, s, h = x.shape
    T = b * s
    F = W1.shape[-1]
    E = Wr.shape[-1]
    xf = x.reshape(T, h)

    NB = (_TOPK * T) // _BM + E
    P = NB * _BM

    pos2d, wk2d, bmap2d, xidx2d, nact2d = _router(xf, Wr, br, NB)
    pos = pos2d.reshape(-1)
    w_flat = wk2d.reshape(-1)
    bmap = bmap2d.reshape(-1)
    xidx = xidx2d.reshape(-1)
    nact = nact2d.reshape(-1)

    tok_base = jnp.arange(P, dtype=jnp.int32) % T

    xs, w_sorted = _dispatch(xf, pos, w_flat, tok_base, P, h)

    ys = _grouped_ffn(nact, xidx, bmap, xs, W1, b1[:, None, :], W2,
                      b2[:, None, :], w_sorted[:, None], NB, P, h, F)

    out = _combine(ys, pos, T, h)
    return out.reshape(b, s, h)
